# baseline (speedup 1.0000x reference)
import jax
import jax.numpy as jnp
from jax.experimental import pallas as pl
from jax.experimental.pallas import tpu as pltpu

_BLOCK_ROWS = 512


def _gap_body(x_ref, o_ref):
    inv = 1.0 / x_ref.shape[1]
    o_ref[...] = jnp.sum(x_ref[...], axis=1, keepdims=True) * inv


def kernel(x):
    b, c, h, w = x.shape
    rows = b * c
    hw = h * w
    x2 = x.reshape(rows, hw)
    grid = (rows // _BLOCK_ROWS,)
    out = pl.pallas_call(
        _gap_body,
        out_shape=jax.ShapeDtypeStruct((rows, 1), x.dtype),
        grid=grid,
        in_specs=[pl.BlockSpec((_BLOCK_ROWS, hw), lambda i: (i, 0))],
        out_specs=pl.BlockSpec((_BLOCK_ROWS, 1), lambda i: (i, 0)),
        compiler_params=pltpu.CompilerParams(
            dimension_semantics=("arbitrary",),
        ),
    )(x2)
    return out.reshape(b, c)


# speedup vs baseline: 9.5846x; 9.5846x over previous
import jax
import jax.numpy as jnp
from jax.experimental import pallas as pl
from jax.experimental.pallas import tpu as pltpu


def _gap_body(x_ref, o_ref):
    inv = 1.0 / x_ref.shape[0]
    o_ref[0, ...] = jnp.sum(x_ref[...], axis=0, keepdims=True) * inv


def kernel(x):
    b, c, h, w = x.shape
    hw = h * w
    x2 = jnp.transpose(x, (0, 2, 3, 1)).reshape(b * hw, c)
    out = pl.pallas_call(
        _gap_body,
        out_shape=jax.ShapeDtypeStruct((b, 1, c), x.dtype),
        grid=(b,),
        in_specs=[pl.BlockSpec((hw, c), lambda i: (i, 0))],
        out_specs=pl.BlockSpec((1, 1, c), lambda i: (i, 0, 0)),
        compiler_params=pltpu.CompilerParams(
            dimension_semantics=("arbitrary",),
            vmem_limit_bytes=50 * 1024 * 1024,
        ),
    )(x2)
    return out.reshape(b, c)
